# baseline (speedup 1.0000x reference)
import math

import jax
import jax.numpy as jnp
from jax import lax
from jax.experimental import pallas as pl

NUM_TOP = 16
TWO_PI = float(2.0 * math.pi)

_ROUND_MAGIC = 12582912.0
_COS_POLY = (0.9999992109801167, -19.73898036851825, 64.92865763797205,
             -85.27162288910772, 58.79049502483567, -21.071106195169147)
_EXP2_POLY = (1.000000075499126, 0.6931472067117411, 0.24022107337696416,
              0.055503272118169404, 0.009676038065012417,
              0.0013400433122416943)


def _cos2pi(u):
    n = jnp.floor(u + 0.5)
    d = u - n
    v = d * d
    p = jnp.float32(_COS_POLY[-1])
    for a in _COS_POLY[-2::-1]:
        p = p * v + jnp.float32(a)
    return p


def _body(xT_ref, col_ref, pos_ref, scl_ref, rot_ref, wcx_ref, wcy_ref, out_ref):
    i = pl.program_id(0)

    @pl.when(i == 0)
    def _init():
        out_ref[...] = jnp.zeros_like(out_ref)

    GB, F = wcx_ref.shape
    f_scale = 1024.0 / float(F)

    iota_f = lax.broadcasted_iota(jnp.int32, (GB, F), 1).astype(jnp.float32)

    Q = F // 4
    iota_q = iota_f[:, :Q]

    def topk(wc):
        a = jnp.abs(wc)
        enc_src = 2.0 * iota_f + jnp.where(wc < 0.0, 1.0, 0.0)
        big = jnp.float32(2.0 * F + 2.0)
        vs = [a[:, i * Q:(i + 1) * Q] for i in range(4)]
        es = [enc_src[:, i * Q:(i + 1) * Q] for i in range(4)]

        def ce(i, j):
            swap = (vs[j] > vs[i]) | ((vs[j] == vs[i]) & (es[j] < es[i]))
            vs[i], vs[j] = (jnp.where(swap, vs[j], vs[i]),
                            jnp.where(swap, vs[i], vs[j]))
            es[i], es[j] = (jnp.where(swap, es[j], es[i]),
                            jnp.where(swap, es[i], es[j]))

        ce(0, 1); ce(2, 3); ce(0, 2); ce(1, 3); ce(1, 2)
        V1, V2, V3, V4 = vs
        E1, E2, E3 = es[0], es[1], es[2]
        E4 = es[3]
        cs, fs = [], []
        for _ in range(NUM_TOP):
            m = jnp.max(V1, axis=1, keepdims=True)
            e = jnp.min(jnp.where(V1 == m, E1, big), axis=1, keepdims=True)
            idx = jnp.floor(e * 0.5)
            sign = e - 2.0 * idx
            cs.append(m * (1.0 - 2.0 * sign))
            fs.append(idx * f_scale)
            gq = jnp.floor(idx * jnp.float32(1.0 / Q))
            gidx = idx - gq * jnp.float32(Q)
            eq = iota_q == gidx
            V1 = jnp.where(eq, V2, V1)
            E1 = jnp.where(eq, E2, E1)
            V2 = jnp.where(eq, V3, V2)
            E2 = jnp.where(eq, E3, E2)
            V3 = jnp.where(eq, V4, V3)
            E3 = jnp.where(eq, E4, E3)
            V4 = jnp.where(eq, -1.0, V4)
        return cs, fs

    cxs, fxs = topk(wcx_ref[...])
    cys, fys = topk(wcy_ref[...])

    x0 = xT_ref[0:1, :]
    x1 = xT_ref[1:2, :]
    p0 = pos_ref[:, 0:1]
    p1 = pos_ref[:, 1:2]
    rot = rot_ref[:, 0:1]
    c = jnp.cos(rot)
    s = jnp.sin(rot)
    rel0 = x0 - p0
    rel1 = x1 - p1
    tx = c * rel0 + s * rel1
    ty = -s * rel0 + c * rel1
    sx = scl_ref[:, 0:1]
    sy = scl_ref[:, 1:2]
    txs = tx * sx
    tys = ty * sy
    nhl2e = jnp.float32(-0.5 * 1.4426950408889634)
    w = nhl2e * (txs * txs) + nhl2e * (tys * tys)
    env = jnp.exp2(w)

    def wave_sum(t, cs, fs):
        acc = jnp.zeros_like(t)
        for k in range(NUM_TOP):
            u = fs[k] * t
            n = jnp.floor(u + 0.5)
            d = u - n
            v = d * d
            p = jnp.float32(_COS_POLY[-1])
            for a in _COS_POLY[-2::-1]:
                p = p * v + jnp.float32(a)
            acc = acc + cs[k] * p
        return acc

    wx = wave_sum(tx, cxs, fxs)
    wy = wave_sum(ty, cys, fys)
    w = env * wx * wy

    for ch in range(3):
        out_ref[ch:ch + 1, :] += jnp.sum(w * col_ref[:, ch:ch + 1], axis=0,
                                         keepdims=True)


def kernel(x, gaussian_colors, gaussian_positions, gaussian_scales,
           gaussian_rotations, wave_coefficients):
    N = x.shape[0]
    G = gaussian_positions.shape[0]
    F = wave_coefficients.shape[2]

    GB = 400 if G % 400 == 0 else G
    num_blocks = G // GB

    wcx = wave_coefficients[:, 0, :]
    wcy = wave_coefficients[:, 1, :]
    xT = x.T

    out = pl.pallas_call(
        _body,
        grid=(num_blocks,),
        in_specs=[
            pl.BlockSpec((2, N), lambda i: (0, 0)),
            pl.BlockSpec((GB, 3), lambda i: (i, 0)),
            pl.BlockSpec((GB, 2), lambda i: (i, 0)),
            pl.BlockSpec((GB, 2), lambda i: (i, 0)),
            pl.BlockSpec((GB, 1), lambda i: (i, 0)),
            pl.BlockSpec((GB, F), lambda i: (i, 0)),
            pl.BlockSpec((GB, F), lambda i: (i, 0)),
        ],
        out_specs=pl.BlockSpec((3, N), lambda i: (0, 0)),
        out_shape=jax.ShapeDtypeStruct((3, N), jnp.float32),
    )(xT, gaussian_colors, gaussian_positions, gaussian_scales,
      gaussian_rotations, wcx, wcy)
    return out.T


# speedup vs baseline: 1.0848x; 1.0848x over previous
import math

import jax
import jax.numpy as jnp
from jax import lax
from jax.experimental import pallas as pl

NUM_TOP = 16
TWO_PI = float(2.0 * math.pi)

_ROUND_MAGIC = 12582912.0
_COS_POLY = (0.9999992109801167, -19.73898036851825, 64.92865763797205,
             -85.27162288910772, 58.79049502483567, -21.071106195169147)
_EXP2_POLY = (1.000000075499126, 0.6931472067117411, 0.24022107337696416,
              0.055503272118169404, 0.009676038065012417,
              0.0013400433122416943)


def _cos2pi(u):
    n = jnp.floor(u + 0.5)
    d = u - n
    v = d * d
    p = jnp.float32(_COS_POLY[-1])
    for a in _COS_POLY[-2::-1]:
        p = p * v + jnp.float32(a)
    return p


def _body(xT_ref, col_ref, pos_ref, scl_ref, rot_ref, wcx_ref, wcy_ref, out_ref):
    i = pl.program_id(0)

    @pl.when(i == 0)
    def _init():
        out_ref[...] = jnp.zeros_like(out_ref)

    GB, F = wcx_ref.shape
    f_scale = 1024.0 / float(F)

    iota_f = lax.broadcasted_iota(jnp.int32, (GB, F), 1).astype(jnp.float32)

    H = F // 2
    iota_h = iota_f[:, :H]

    def topk(wc):
        a = jnp.abs(wc)
        enc_src = 2.0 * iota_f + jnp.where(wc < 0.0, 1.0, 0.0)
        big = jnp.float32(2.0 * F + 2.0)
        aL, aR = a[:, :H], a[:, H:]
        eL, eR = enc_src[:, :H], enc_src[:, H:]
        pick = aL >= aR
        P = jnp.where(pick, aL, aR)
        Pm = jnp.where(pick, aR, aL)
        E = jnp.where(pick, eL, eR)
        Em = jnp.where(pick, eR, eL)
        cs, fs = [], []
        for _ in range(NUM_TOP):
            m = jnp.max(P, axis=1, keepdims=True)
            e = jnp.min(jnp.where(P == m, E, big), axis=1, keepdims=True)
            idx = jnp.floor(e * 0.5)
            sign = e - 2.0 * idx
            cs.append(m * (1.0 - 2.0 * sign))
            fs.append(idx * f_scale)
            pidx = jnp.where(idx >= H, idx - H, idx)
            eq = iota_h == pidx
            P = jnp.where(eq, Pm, P)
            E = jnp.where(eq, Em, E)
            Pm = jnp.where(eq, -1.0, Pm)
        return cs, fs

    cxs, fxs = topk(wcx_ref[...])
    cys, fys = topk(wcy_ref[...])

    x0 = xT_ref[0:1, :]
    x1 = xT_ref[1:2, :]
    p0 = pos_ref[:, 0:1]
    p1 = pos_ref[:, 1:2]
    rot = rot_ref[:, 0:1]
    c = jnp.cos(rot)
    s = jnp.sin(rot)
    rel0 = x0 - p0
    rel1 = x1 - p1
    tx = c * rel0 + s * rel1
    ty = -s * rel0 + c * rel1
    sx = scl_ref[:, 0:1]
    sy = scl_ref[:, 1:2]
    txs = tx * sx
    tys = ty * sy
    nhl2e = jnp.float32(-0.5 * 1.4426950408889634)
    w = nhl2e * (txs * txs) + nhl2e * (tys * tys)
    env = jnp.exp2(w)

    def wave_sum(t, cs, fs):
        acc = jnp.zeros_like(t)
        for k in range(NUM_TOP):
            u = fs[k] * t
            n = jnp.floor(u + 0.5)
            d = u - n
            v = d * d
            p = jnp.float32(_COS_POLY[-1])
            for a in _COS_POLY[-2::-1]:
                p = p * v + jnp.float32(a)
            acc = acc + cs[k] * p
        return acc

    wx = wave_sum(tx, cxs, fxs)
    wy = wave_sum(ty, cys, fys)
    w = env * wx * wy

    for ch in range(3):
        out_ref[ch:ch + 1, :] += jnp.sum(w * col_ref[:, ch:ch + 1], axis=0,
                                         keepdims=True)


def kernel(x, gaussian_colors, gaussian_positions, gaussian_scales,
           gaussian_rotations, wave_coefficients):
    N = x.shape[0]
    G = gaussian_positions.shape[0]
    F = wave_coefficients.shape[2]

    GB = 400 if G % 400 == 0 else G
    num_blocks = G // GB

    wcx = wave_coefficients[:, 0, :]
    wcy = wave_coefficients[:, 1, :]
    xT = x.T

    out = pl.pallas_call(
        _body,
        grid=(num_blocks,),
        in_specs=[
            pl.BlockSpec((2, N), lambda i: (0, 0)),
            pl.BlockSpec((GB, 3), lambda i: (i, 0)),
            pl.BlockSpec((GB, 2), lambda i: (i, 0)),
            pl.BlockSpec((GB, 2), lambda i: (i, 0)),
            pl.BlockSpec((GB, 1), lambda i: (i, 0)),
            pl.BlockSpec((GB, F), lambda i: (i, 0)),
            pl.BlockSpec((GB, F), lambda i: (i, 0)),
        ],
        out_specs=pl.BlockSpec((3, N), lambda i: (0, 0)),
        out_shape=jax.ShapeDtypeStruct((3, N), jnp.float32),
    )(xT, gaussian_colors, gaussian_positions, gaussian_scales,
      gaussian_rotations, wcx, wcy)
    return out.T
